# Initial kernel scaffold; baseline (speedup 1.0000x reference)
import jax
import jax.numpy as jnp
from jax.experimental import pallas as pl


def kernel(x, Wr, W_self, b_enc, W_fc, b_fc, W_dir, b_dir, W_cont, b_cont, edge_index, edge_type, nest_tensor, food_tensor):
    raise NotImplementedError("write your pallas kernel here")


# speedup vs baseline: 30.8182x; 30.8182x over previous
import jax
import jax.numpy as jnp
from jax import lax
from jax.experimental import pallas as pl
from jax.experimental.pallas import tpu as pltpu
from jax.experimental.pallas import tpu_sc as plsc

NC = 2
NS = 16
NW = NC * NS
L = 16

N = 10000
E = 320000
D = 128
R = 4
B = 1024
U = 2 * B
QROWS = U * R
QROWS_C = U * 2
TRASH = QROWS_C
SROWS_C = QROWS_C + 128
NPAD = 10240
EPS = E // NS
NHALF = 5
NSTRIP = 2
STRIP = EPS // (NHALF * NSTRIP)
CAP = 4096
KB = 64


def _sc_body(src_hbm, dst_hbm, typ_hbm, nest_hbm, food_hbm, x_hbm,
             sb_hbm, dg_hbm, xu_hbm,
             es_v, ed_v, et_v, slotmap, u_v, srcc, combc, comb2d, rows_v,
             ones_v, slb_v, s_sh, deg_sh, sem_g, sem_s, sem_d):
  cid = lax.axis_index("c")
  sid = lax.axis_index("s")
  wid = sid * NC + cid
  i16 = lax.broadcasted_iota(jnp.int32, (L,), 0)

  pltpu.sync_copy(nest_hbm, u_v.at[pl.ds(0, B)])
  pltpu.sync_copy(food_hbm, u_v.at[pl.ds(B, B)])

  def zrows(i, c):
    for cc in range(D // L):
      rows_v[i, pl.ds(cc * L, L)] = jnp.zeros((L,), jnp.float32)
      ones_v[i, pl.ds(cc * L, L)] = jnp.full((L,), 1.0, jnp.float32)
    return c
  lax.fori_loop(0, KB, zrows, 0)

  s0 = sid * (SROWS_C // NS)
  for off, nn in ((0, KB), (KB, KB), (2 * KB, KB), (3 * KB, KB),
                  (4 * KB, 8)):
    pltpu.sync_copy(rows_v.at[pl.ds(0, nn)], s_sh.at[pl.ds(s0 + off, nn)])
    pltpu.sync_copy(rows_v.at[pl.ds(0, nn)], deg_sh.at[pl.ds(s0 + off, nn)])

  def sm_init(g, c):
    slotmap[pl.ds(g * L, L)] = jnp.full((L,), -1, jnp.int32)
    return c
  lax.fori_loop(0, NPAD // L, sm_init, 0)

  def sm_scat(g, c):
    idx = u_v[pl.ds(g * L, L)]
    plsc.store_scatter(slotmap, [idx], g * L + i16)
    return c
  lax.fori_loop(0, U // L, sm_scat, 0)

  plsc.subcore_barrier()

  for half in range(NHALF):
    def strip_body(s, cnt):
      base = sid * EPS + (half * NSTRIP + s) * STRIP
      pltpu.sync_copy(src_hbm.at[pl.ds(base, STRIP)], es_v)
      pltpu.sync_copy(dst_hbm.at[pl.ds(base, STRIP)], ed_v)
      pltpu.sync_copy(typ_hbm.at[pl.ds(base, STRIP)], et_v)

      def grp(g, cnt):
        d16 = ed_v[pl.ds(g * L, L)]
        t16 = et_v[pl.ds(g * L, L)]
        s16 = es_v[pl.ds(g * L, L)]
        sl = plsc.load_gather(slotmap, [d16])
        m = jnp.logical_and(sl >= 0,
                            lax.shift_right_logical(t16, 1) == cid)
        comb = sl * 2 + lax.bitwise_and(t16, 1)
        plsc.store_compressed(srcc.at[pl.ds(cnt, L)], s16, mask=m)
        plsc.store_compressed(combc.at[pl.ds(cnt, L)], comb, mask=m)
        pc = plsc.all_reduce_population_count(m)
        return cnt + pc.max().astype(jnp.int32)
      return lax.fori_loop(0, STRIP // L, grp, cnt)
    cnt = lax.fori_loop(0, NSTRIP, strip_body, jnp.int32(0))

    def padt(t, c):
      srcc[pl.ds(cnt + t * L, L)] = jnp.zeros((L,), jnp.int32)
      combc[pl.ds(cnt + t * L, L)] = jnp.full((L,), TRASH, jnp.int32)
      return c
    lax.fori_loop(0, KB // L, padt, 0)
    nbat = lax.shift_right_logical(cnt + KB - 1, 6)

    def repack(g, c):
      comb2d[lax.shift_right_logical(g, 2),
             pl.ds(lax.bitwise_and(g, 3) * L, L)] = combc[pl.ds(g * L, L)]
      return c
    lax.fori_loop(0, nbat * (KB // L), repack, 0)

    def bat(bi, c):
      pltpu.async_copy(x_hbm.at[srcc.at[pl.ds(bi * KB, KB)]], rows_v,
                       sem_g).wait()
      c1 = pltpu.async_copy(rows_v, s_sh.at[comb2d.at[bi]], sem_s, add=True)
      c2 = pltpu.async_copy(ones_v, deg_sh.at[comb2d.at[bi]], sem_d,
                            add=True)
      c1.wait()
      c2.wait()
      return c
    lax.fori_loop(0, nbat, bat, 0)

  plsc.subcore_barrier()

  qbase = sid * (QROWS_C // NS)
  def slb_b(g, c):
    q = qbase + g * L + i16
    rp = lax.shift_right_logical(q, 11)
    j = lax.bitwise_and(q, 2047)
    uj = plsc.load_gather(u_v, [j])
    sl = plsc.load_gather(slotmap, [uj])
    slb_v[pl.ds(g * L, L)] = sl * 2 + rp
    return c
  lax.fori_loop(0, (QROWS_C // NS) // L, slb_b, 0)

  for k in range(4):
    idxs = slb_v.at[pl.ds(k * KB, KB)]
    pltpu.async_copy(s_sh.at[idxs], rows_v, sem_g).wait()
    pltpu.sync_copy(rows_v,
                    sb_hbm.at[pl.ds(cid * QROWS_C + qbase + k * KB, KB)])
    pltpu.async_copy(deg_sh.at[idxs], rows_v, sem_d).wait()
    pltpu.sync_copy(rows_v,
                    dg_hbm.at[pl.ds(cid * QROWS_C + qbase + k * KB, KB)])

  for k in range(2):
    xo = wid * (U // NW) + k * 32
    pltpu.async_copy(x_hbm.at[u_v.at[pl.ds(xo, 32)]],
                     rows_v.at[pl.ds(0, 32)], sem_g).wait()
    pltpu.sync_copy(rows_v.at[pl.ds(0, 32)], xu_hbm.at[pl.ds(xo, 32)])


_sc_gather_scatter = pl.kernel(
    _sc_body,
    out_type=(
        jax.ShapeDtypeStruct((QROWS, D), jnp.float32),
        jax.ShapeDtypeStruct((QROWS, D), jnp.float32),
        jax.ShapeDtypeStruct((U, D), jnp.float32),
    ),
    mesh=plsc.VectorSubcoreMesh(core_axis_name="c", subcore_axis_name="s"),
    compiler_params=pltpu.CompilerParams(needs_layout_passes=False),
    scratch_types=[
        pltpu.VMEM((STRIP,), jnp.int32),
        pltpu.VMEM((STRIP,), jnp.int32),
        pltpu.VMEM((STRIP,), jnp.int32),
        pltpu.VMEM((NPAD,), jnp.int32),
        pltpu.VMEM((U,), jnp.int32),
        pltpu.VMEM((CAP,), jnp.int32),
        pltpu.VMEM((CAP,), jnp.int32),
        pltpu.VMEM((CAP // KB, KB), jnp.int32),
        pltpu.VMEM((KB, D), jnp.float32),
        pltpu.VMEM((KB, D), jnp.float32),
        pltpu.VMEM((QROWS_C // NS,), jnp.int32),
        pltpu.VMEM_SHARED((SROWS_C, D), jnp.float32),
        pltpu.VMEM_SHARED((SROWS_C, D), jnp.float32),
        pltpu.SemaphoreType.DMA,
        pltpu.SemaphoreType.DMA,
        pltpu.SemaphoreType.DMA,
    ],
)


def _head_body(sb_ref, dg_ref, xu_ref, wr_ref, wself_ref, benc_ref, wfc_ref,
               bfc_ref, wdir_ref, bdir_ref, wcont_ref, bcont_ref,
               logits_ref, mu_ref, logvar_ref):
  f32 = jnp.float32
  hi = jax.lax.Precision.HIGHEST
  scale = 1.0 / jnp.maximum(dg_ref[:, 0:1], 1.0)
  sn = sb_ref[...] * scale
  contrib = jnp.zeros((U, D), f32)
  for r in range(R):
    contrib = contrib + jnp.dot(sn[r * U:(r + 1) * U], wr_ref[r],
                                preferred_element_type=f32, precision=hi)
  node = contrib + jnp.dot(xu_ref[...], wself_ref[...],
                           preferred_element_type=f32, precision=hi)
  node = jnp.maximum(node + benc_ref[...], 0.0)
  hin = jnp.concatenate([node[:B], node[B:]], axis=1)
  h = jnp.tanh(jnp.dot(hin, wfc_ref[...], preferred_element_type=f32,
                       precision=hi) + bfc_ref[...])
  logits_ref[...] = jnp.dot(h, wdir_ref[...], preferred_element_type=f32,
                            precision=hi) + bdir_ref[...]
  cont = jnp.dot(h, wcont_ref[...], preferred_element_type=f32,
                 precision=hi) + bcont_ref[...]
  mu_ref[...] = cont[:, 0:1]
  logvar_ref[...] = cont[:, 1:2]


_head = pl.pallas_call(
    _head_body,
    out_shape=(
        jax.ShapeDtypeStruct((B, R), jnp.float32),
        jax.ShapeDtypeStruct((B, 1), jnp.float32),
        jax.ShapeDtypeStruct((B, 1), jnp.float32),
    ),
)


def kernel(x, Wr, W_self, b_enc, W_fc, b_fc, W_dir, b_dir, W_cont, b_cont,
           edge_index, edge_type, nest_tensor, food_tensor):
  src = edge_index[0].astype(jnp.int32)
  dst = edge_index[1].astype(jnp.int32)
  typ = edge_type.astype(jnp.int32)
  sb, dg, xu = _sc_gather_scatter(src, dst, typ,
                                  nest_tensor.astype(jnp.int32),
                                  food_tensor.astype(jnp.int32), x)
  logits, mu, logvar = _head(
      sb, dg, xu, Wr, W_self, b_enc.reshape(1, D), W_fc, b_fc.reshape(1, -1),
      W_dir, b_dir.reshape(1, -1), W_cont, b_cont.reshape(1, -1))
  return logits, mu, logvar


# speedup vs baseline: 31.5592x; 1.0240x over previous
import jax
import jax.numpy as jnp
from jax import lax
from jax.experimental import pallas as pl
from jax.experimental.pallas import tpu as pltpu
from jax.experimental.pallas import tpu_sc as plsc

NC = 2
NS = 16
NW = NC * NS
L = 16

N = 10000
E = 320000
D = 128
R = 4
B = 1024
U = 2 * B
QROWS = U * R
QROWS_C = U * 2
TRASH = QROWS_C
SROWS_C = QROWS_C + 128
NPAD = 10240
EPS = E // NS
NHALF = 5
NSTRIP = 2
STRIP = EPS // (NHALF * NSTRIP)
CAP = 4096
KB = 64


def _sc_body(src_hbm, dst_hbm, typ_hbm, nest_hbm, food_hbm, x_hbm,
             sb_hbm, dg_hbm, xu_hbm,
             es_v, ed_v, et_v, slotmap, u_v, srcc, combc, comb2d, rows2,
             ones_v, slb_v, s_sh, deg_sh, sem_g, sem_s, sem_d):
  rows_v = rows2.at[0]
  cid = lax.axis_index("c")
  sid = lax.axis_index("s")
  wid = sid * NC + cid
  i16 = lax.broadcasted_iota(jnp.int32, (L,), 0)

  pltpu.sync_copy(nest_hbm, u_v.at[pl.ds(0, B)])
  pltpu.sync_copy(food_hbm, u_v.at[pl.ds(B, B)])

  def zrows(i, c):
    for cc in range(D // L):
      rows2[0, i, pl.ds(cc * L, L)] = jnp.zeros((L,), jnp.float32)
      ones_v[i, pl.ds(cc * L, L)] = jnp.full((L,), 1.0, jnp.float32)
    return c
  lax.fori_loop(0, KB, zrows, 0)

  s0 = sid * (SROWS_C // NS)
  for off, nn in ((0, KB), (KB, KB), (2 * KB, KB), (3 * KB, KB),
                  (4 * KB, 8)):
    pltpu.sync_copy(rows_v.at[pl.ds(0, nn)], s_sh.at[pl.ds(s0 + off, nn)])
    pltpu.sync_copy(rows_v.at[pl.ds(0, nn)], deg_sh.at[pl.ds(s0 + off, nn)])

  def sm_init(g, c):
    slotmap[pl.ds(g * L, L)] = jnp.full((L,), -1, jnp.int32)
    return c
  lax.fori_loop(0, NPAD // L, sm_init, 0)

  def sm_scat(g, c):
    idx = u_v[pl.ds(g * L, L)]
    plsc.store_scatter(slotmap, [idx], g * L + i16)
    return c
  lax.fori_loop(0, U // L, sm_scat, 0)

  plsc.subcore_barrier()

  for half in range(NHALF):
    def strip_body(s, cnt):
      base = sid * EPS + (half * NSTRIP + s) * STRIP
      pltpu.sync_copy(src_hbm.at[pl.ds(base, STRIP)], es_v)
      pltpu.sync_copy(dst_hbm.at[pl.ds(base, STRIP)], ed_v)
      pltpu.sync_copy(typ_hbm.at[pl.ds(base, STRIP)], et_v)

      def grp(g, cnt):
        d16 = ed_v[pl.ds(g * L, L)]
        t16 = et_v[pl.ds(g * L, L)]
        s16 = es_v[pl.ds(g * L, L)]
        sl = plsc.load_gather(slotmap, [d16])
        m = jnp.logical_and(sl >= 0,
                            lax.shift_right_logical(t16, 1) == cid)
        comb = sl * 2 + lax.bitwise_and(t16, 1)
        plsc.store_compressed(srcc.at[pl.ds(cnt, L)], s16, mask=m)
        plsc.store_compressed(combc.at[pl.ds(cnt, L)], comb, mask=m)
        pc = plsc.all_reduce_population_count(m)
        return cnt + pc.max().astype(jnp.int32)
      return lax.fori_loop(0, STRIP // L, grp, cnt)
    cnt = lax.fori_loop(0, NSTRIP, strip_body, jnp.int32(0))

    def padt(t, c):
      srcc[pl.ds(cnt + t * L, L)] = jnp.zeros((L,), jnp.int32)
      combc[pl.ds(cnt + t * L, L)] = jnp.full((L,), TRASH, jnp.int32)
      return c
    lax.fori_loop(0, KB // L, padt, 0)
    nbat = lax.shift_right_logical(cnt + KB - 1, 6)

    def repack(g, c):
      comb2d[lax.shift_right_logical(g, 2),
             pl.ds(lax.bitwise_and(g, 3) * L, L)] = combc[pl.ds(g * L, L)]
      return c
    lax.fori_loop(0, nbat * (KB // L), repack, 0)

    @pl.when(nbat > 0)
    def _prime():
      pltpu.async_copy(x_hbm.at[srcc.at[pl.ds(0, KB)]], rows2.at[0], sem_g)

    def bat(bi, c):
      b = lax.bitwise_and(bi, 1)
      cur = rows2.at[b]
      pltpu.make_async_copy(x_hbm.at[srcc.at[pl.ds(bi * KB, KB)]], cur,
                            sem_g).wait()

      @pl.when(bi >= 1)
      def _drain_prev():
        prev = rows2.at[1 - b]
        pltpu.make_async_copy(prev, s_sh.at[comb2d.at[bi - 1]], sem_s).wait()
        pltpu.make_async_copy(ones_v, deg_sh.at[comb2d.at[bi - 1]],
                              sem_d).wait()

      @pl.when(bi + 1 < nbat)
      def _next_gather():
        pltpu.async_copy(x_hbm.at[srcc.at[pl.ds((bi + 1) * KB, KB)]],
                         rows2.at[1 - b], sem_g)

      pltpu.async_copy(cur, s_sh.at[comb2d.at[bi]], sem_s, add=True)
      pltpu.async_copy(ones_v, deg_sh.at[comb2d.at[bi]], sem_d, add=True)
      return c
    lax.fori_loop(0, nbat, bat, 0)

    @pl.when(nbat > 0)
    def _drain_last():
      bl = nbat - 1
      pltpu.make_async_copy(rows2.at[lax.bitwise_and(bl, 1)],
                            s_sh.at[comb2d.at[bl]], sem_s).wait()
      pltpu.make_async_copy(ones_v, deg_sh.at[comb2d.at[bl]], sem_d).wait()

  plsc.subcore_barrier()

  qbase = sid * (QROWS_C // NS)
  def slb_b(g, c):
    q = qbase + g * L + i16
    rp = lax.shift_right_logical(q, 11)
    j = lax.bitwise_and(q, 2047)
    uj = plsc.load_gather(u_v, [j])
    sl = plsc.load_gather(slotmap, [uj])
    slb_v[pl.ds(g * L, L)] = sl * 2 + rp
    return c
  lax.fori_loop(0, (QROWS_C // NS) // L, slb_b, 0)

  for k in range(4):
    idxs = slb_v.at[pl.ds(k * KB, KB)]
    pltpu.async_copy(s_sh.at[idxs], rows_v, sem_g).wait()
    pltpu.sync_copy(rows_v,
                    sb_hbm.at[pl.ds(cid * QROWS_C + qbase + k * KB, KB)])
    pltpu.async_copy(deg_sh.at[idxs], rows_v, sem_d).wait()
    pltpu.sync_copy(rows_v,
                    dg_hbm.at[pl.ds(cid * QROWS_C + qbase + k * KB, KB)])

  for k in range(2):
    xo = wid * (U // NW) + k * 32
    pltpu.async_copy(x_hbm.at[u_v.at[pl.ds(xo, 32)]],
                     rows_v.at[pl.ds(0, 32)], sem_g).wait()
    pltpu.sync_copy(rows_v.at[pl.ds(0, 32)], xu_hbm.at[pl.ds(xo, 32)])


_sc_gather_scatter = pl.kernel(
    _sc_body,
    out_type=(
        jax.ShapeDtypeStruct((QROWS, D), jnp.float32),
        jax.ShapeDtypeStruct((QROWS, D), jnp.float32),
        jax.ShapeDtypeStruct((U, D), jnp.float32),
    ),
    mesh=plsc.VectorSubcoreMesh(core_axis_name="c", subcore_axis_name="s"),
    compiler_params=pltpu.CompilerParams(needs_layout_passes=False),
    scratch_types=[
        pltpu.VMEM((STRIP,), jnp.int32),
        pltpu.VMEM((STRIP,), jnp.int32),
        pltpu.VMEM((STRIP,), jnp.int32),
        pltpu.VMEM((NPAD,), jnp.int32),
        pltpu.VMEM((U,), jnp.int32),
        pltpu.VMEM((CAP,), jnp.int32),
        pltpu.VMEM((CAP,), jnp.int32),
        pltpu.VMEM((CAP // KB, KB), jnp.int32),
        pltpu.VMEM((2, KB, D), jnp.float32),
        pltpu.VMEM((KB, D), jnp.float32),
        pltpu.VMEM((QROWS_C // NS,), jnp.int32),
        pltpu.VMEM_SHARED((SROWS_C, D), jnp.float32),
        pltpu.VMEM_SHARED((SROWS_C, D), jnp.float32),
        pltpu.SemaphoreType.DMA,
        pltpu.SemaphoreType.DMA,
        pltpu.SemaphoreType.DMA,
    ],
)


def _head_body(sb_ref, dg_ref, xu_ref, wr_ref, wself_ref, benc_ref, wfc_ref,
               bfc_ref, wdir_ref, bdir_ref, wcont_ref, bcont_ref,
               logits_ref, mu_ref, logvar_ref):
  f32 = jnp.float32
  hi = jax.lax.Precision.HIGHEST
  scale = 1.0 / jnp.maximum(dg_ref[:, 0:1], 1.0)
  sn = sb_ref[...] * scale
  contrib = jnp.zeros((U, D), f32)
  for r in range(R):
    contrib = contrib + jnp.dot(sn[r * U:(r + 1) * U], wr_ref[r],
                                preferred_element_type=f32, precision=hi)
  node = contrib + jnp.dot(xu_ref[...], wself_ref[...],
                           preferred_element_type=f32, precision=hi)
  node = jnp.maximum(node + benc_ref[...], 0.0)
  hin = jnp.concatenate([node[:B], node[B:]], axis=1)
  h = jnp.tanh(jnp.dot(hin, wfc_ref[...], preferred_element_type=f32,
                       precision=hi) + bfc_ref[...])
  logits_ref[...] = jnp.dot(h, wdir_ref[...], preferred_element_type=f32,
                            precision=hi) + bdir_ref[...]
  cont = jnp.dot(h, wcont_ref[...], preferred_element_type=f32,
                 precision=hi) + bcont_ref[...]
  mu_ref[...] = cont[:, 0:1]
  logvar_ref[...] = cont[:, 1:2]


_head = pl.pallas_call(
    _head_body,
    out_shape=(
        jax.ShapeDtypeStruct((B, R), jnp.float32),
        jax.ShapeDtypeStruct((B, 1), jnp.float32),
        jax.ShapeDtypeStruct((B, 1), jnp.float32),
    ),
)


def kernel(x, Wr, W_self, b_enc, W_fc, b_fc, W_dir, b_dir, W_cont, b_cont,
           edge_index, edge_type, nest_tensor, food_tensor):
  src = edge_index[0].astype(jnp.int32)
  dst = edge_index[1].astype(jnp.int32)
  typ = edge_type.astype(jnp.int32)
  sb, dg, xu = _sc_gather_scatter(src, dst, typ,
                                  nest_tensor.astype(jnp.int32),
                                  food_tensor.astype(jnp.int32), x)
  logits, mu, logvar = _head(
      sb, dg, xu, Wr, W_self, b_enc.reshape(1, D), W_fc, b_fc.reshape(1, -1),
      W_dir, b_dir.reshape(1, -1), W_cont, b_cont.reshape(1, -1))
  return logits, mu, logvar


# speedup vs baseline: 32.0528x; 1.0156x over previous
import jax
import jax.numpy as jnp
from jax import lax
from jax.experimental import pallas as pl
from jax.experimental.pallas import tpu as pltpu
from jax.experimental.pallas import tpu_sc as plsc

NC = 2
NS = 16
NW = NC * NS
L = 16

N = 10000
E = 320000
D = 128
R = 4
B = 1024
U = 2 * B
QROWS = U * R
QROWS_C = U * 2
TRASH = QROWS_C
SROWS_C = QROWS_C + 128
NPAD = 10240
EPS = E // NS
NHALF = 5
NSTRIP = 2
STRIP = EPS // (NHALF * NSTRIP)
CAP = 4096
KB = 64


def _sc_body(src_hbm, dst_hbm, typ_hbm, nest_hbm, food_hbm, x_hbm,
             sb_hbm, dg_hbm, xu_hbm,
             es_v, ed_v, et_v, slotmap, u_v, srcc, combc, comb2d, rows2,
             ones_v, slb_v, s_sh, deg_sh, sem_g, sem_s, sem_d):
  rows_v = rows2.at[0]
  cid = lax.axis_index("c")
  sid = lax.axis_index("s")
  wid = sid * NC + cid
  i16 = lax.broadcasted_iota(jnp.int32, (L,), 0)

  pltpu.sync_copy(nest_hbm, u_v.at[pl.ds(0, B)])
  pltpu.sync_copy(food_hbm, u_v.at[pl.ds(B, B)])

  def zrows(i, c):
    for cc in range(D // L):
      rows2[0, i, pl.ds(cc * L, L)] = jnp.zeros((L,), jnp.float32)
      ones_v[i, pl.ds(cc * L, L)] = jnp.full((L,), 1.0, jnp.float32)
    return c
  lax.fori_loop(0, KB, zrows, 0)

  s0 = sid * (SROWS_C // NS)
  zcopies = []
  for off, nn in ((0, KB), (KB, KB), (2 * KB, KB), (3 * KB, KB),
                  (4 * KB, 8)):
    zcopies.append(pltpu.async_copy(rows_v.at[pl.ds(0, nn)],
                                    s_sh.at[pl.ds(s0 + off, nn)], sem_s))
    zcopies.append(pltpu.async_copy(rows_v.at[pl.ds(0, nn)],
                                    deg_sh.at[pl.ds(s0 + off, nn)], sem_d))

  def sm_init(g, c):
    slotmap[pl.ds(g * L, L)] = jnp.full((L,), -1, jnp.int32)
    return c
  lax.fori_loop(0, NPAD // L, sm_init, 0)

  def sm_scat(g, c):
    idx = u_v[pl.ds(g * L, L)]
    plsc.store_scatter(slotmap, [idx], g * L + i16)
    return c
  lax.fori_loop(0, U // L, sm_scat, 0)

  for zc in zcopies:
    zc.wait()
  plsc.subcore_barrier()

  for half in range(NHALF):
    def strip_body(s, cnt):
      base = sid * EPS + (half * NSTRIP + s) * STRIP
      e1 = pltpu.async_copy(src_hbm.at[pl.ds(base, STRIP)], es_v, sem_g)
      e2 = pltpu.async_copy(dst_hbm.at[pl.ds(base, STRIP)], ed_v, sem_g)
      e3 = pltpu.async_copy(typ_hbm.at[pl.ds(base, STRIP)], et_v, sem_g)
      e1.wait()
      e2.wait()
      e3.wait()

      def grp(g, cnt):
        d16 = ed_v[pl.ds(g * L, L)]
        t16 = et_v[pl.ds(g * L, L)]
        s16 = es_v[pl.ds(g * L, L)]
        sl = plsc.load_gather(slotmap, [d16])
        m = jnp.logical_and(sl >= 0,
                            lax.shift_right_logical(t16, 1) == cid)
        comb = sl * 2 + lax.bitwise_and(t16, 1)
        plsc.store_compressed(srcc.at[pl.ds(cnt, L)], s16, mask=m)
        plsc.store_compressed(combc.at[pl.ds(cnt, L)], comb, mask=m)
        pc = plsc.all_reduce_population_count(m)
        return cnt + pc.max().astype(jnp.int32)
      return lax.fori_loop(0, STRIP // L, grp, cnt)
    cnt = lax.fori_loop(0, NSTRIP, strip_body, jnp.int32(0))

    def padt(t, c):
      srcc[pl.ds(cnt + t * L, L)] = jnp.zeros((L,), jnp.int32)
      combc[pl.ds(cnt + t * L, L)] = jnp.full((L,), TRASH, jnp.int32)
      return c
    lax.fori_loop(0, KB // L, padt, 0)
    nbat = lax.shift_right_logical(cnt + KB - 1, 6)

    def repack(g, c):
      comb2d[lax.shift_right_logical(g, 2),
             pl.ds(lax.bitwise_and(g, 3) * L, L)] = combc[pl.ds(g * L, L)]
      return c
    lax.fori_loop(0, nbat * (KB // L), repack, 0)

    @pl.when(nbat > 0)
    def _prime():
      pltpu.async_copy(x_hbm.at[srcc.at[pl.ds(0, KB)]], rows2.at[0], sem_g)

    def bat(bi, c):
      b = lax.bitwise_and(bi, 1)
      cur = rows2.at[b]
      pltpu.make_async_copy(x_hbm.at[srcc.at[pl.ds(bi * KB, KB)]], cur,
                            sem_g).wait()

      @pl.when(bi >= 1)
      def _drain_prev():
        prev = rows2.at[1 - b]
        pltpu.make_async_copy(prev, s_sh.at[comb2d.at[bi - 1]], sem_s).wait()
        pltpu.make_async_copy(ones_v, deg_sh.at[comb2d.at[bi - 1]],
                              sem_d).wait()

      @pl.when(bi + 1 < nbat)
      def _next_gather():
        pltpu.async_copy(x_hbm.at[srcc.at[pl.ds((bi + 1) * KB, KB)]],
                         rows2.at[1 - b], sem_g)

      pltpu.async_copy(cur, s_sh.at[comb2d.at[bi]], sem_s, add=True)
      pltpu.async_copy(ones_v, deg_sh.at[comb2d.at[bi]], sem_d, add=True)
      return c
    lax.fori_loop(0, nbat, bat, 0)

    @pl.when(nbat > 0)
    def _drain_last():
      bl = nbat - 1
      pltpu.make_async_copy(rows2.at[lax.bitwise_and(bl, 1)],
                            s_sh.at[comb2d.at[bl]], sem_s).wait()
      pltpu.make_async_copy(ones_v, deg_sh.at[comb2d.at[bl]], sem_d).wait()

  plsc.subcore_barrier()

  qbase = sid * (QROWS_C // NS)
  def slb_b(g, c):
    q = qbase + g * L + i16
    rp = lax.shift_right_logical(q, 11)
    j = lax.bitwise_and(q, 2047)
    uj = plsc.load_gather(u_v, [j])
    sl = plsc.load_gather(slotmap, [uj])
    slb_v[pl.ds(g * L, L)] = sl * 2 + rp
    return c
  lax.fori_loop(0, (QROWS_C // NS) // L, slb_b, 0)

  for k in range(4):
    idxs = slb_v.at[pl.ds(k * KB, KB)]
    pltpu.async_copy(s_sh.at[idxs], rows_v, sem_g).wait()
    pltpu.sync_copy(rows_v,
                    sb_hbm.at[pl.ds(cid * QROWS_C + qbase + k * KB, KB)])
    pltpu.async_copy(deg_sh.at[idxs], rows_v, sem_d).wait()
    pltpu.sync_copy(rows_v,
                    dg_hbm.at[pl.ds(cid * QROWS_C + qbase + k * KB, KB)])

  for k in range(2):
    xo = wid * (U // NW) + k * 32
    pltpu.async_copy(x_hbm.at[u_v.at[pl.ds(xo, 32)]],
                     rows_v.at[pl.ds(0, 32)], sem_g).wait()
    pltpu.sync_copy(rows_v.at[pl.ds(0, 32)], xu_hbm.at[pl.ds(xo, 32)])


_sc_gather_scatter = pl.kernel(
    _sc_body,
    out_type=(
        jax.ShapeDtypeStruct((QROWS, D), jnp.float32),
        jax.ShapeDtypeStruct((QROWS, D), jnp.float32),
        jax.ShapeDtypeStruct((U, D), jnp.float32),
    ),
    mesh=plsc.VectorSubcoreMesh(core_axis_name="c", subcore_axis_name="s"),
    compiler_params=pltpu.CompilerParams(needs_layout_passes=False),
    scratch_types=[
        pltpu.VMEM((STRIP,), jnp.int32),
        pltpu.VMEM((STRIP,), jnp.int32),
        pltpu.VMEM((STRIP,), jnp.int32),
        pltpu.VMEM((NPAD,), jnp.int32),
        pltpu.VMEM((U,), jnp.int32),
        pltpu.VMEM((CAP,), jnp.int32),
        pltpu.VMEM((CAP,), jnp.int32),
        pltpu.VMEM((CAP // KB, KB), jnp.int32),
        pltpu.VMEM((2, KB, D), jnp.float32),
        pltpu.VMEM((KB, D), jnp.float32),
        pltpu.VMEM((QROWS_C // NS,), jnp.int32),
        pltpu.VMEM_SHARED((SROWS_C, D), jnp.float32),
        pltpu.VMEM_SHARED((SROWS_C, D), jnp.float32),
        pltpu.SemaphoreType.DMA,
        pltpu.SemaphoreType.DMA,
        pltpu.SemaphoreType.DMA,
    ],
)


def _head_body(sb_ref, dg_ref, xu_ref, wr_ref, wself_ref, benc_ref, wfc_ref,
               bfc_ref, wdir_ref, bdir_ref, wcont_ref, bcont_ref,
               logits_ref, mu_ref, logvar_ref):
  f32 = jnp.float32
  hi = jax.lax.Precision.HIGHEST
  scale = 1.0 / jnp.maximum(dg_ref[:, 0:1], 1.0)
  sn = sb_ref[...] * scale
  contrib = jnp.zeros((U, D), f32)
  for r in range(R):
    contrib = contrib + jnp.dot(sn[r * U:(r + 1) * U], wr_ref[r],
                                preferred_element_type=f32, precision=hi)
  node = contrib + jnp.dot(xu_ref[...], wself_ref[...],
                           preferred_element_type=f32, precision=hi)
  node = jnp.maximum(node + benc_ref[...], 0.0)
  hin = jnp.concatenate([node[:B], node[B:]], axis=1)
  h = jnp.tanh(jnp.dot(hin, wfc_ref[...], preferred_element_type=f32,
                       precision=hi) + bfc_ref[...])
  logits_ref[...] = jnp.dot(h, wdir_ref[...], preferred_element_type=f32,
                            precision=hi) + bdir_ref[...]
  cont = jnp.dot(h, wcont_ref[...], preferred_element_type=f32,
                 precision=hi) + bcont_ref[...]
  mu_ref[...] = cont[:, 0:1]
  logvar_ref[...] = cont[:, 1:2]


_head = pl.pallas_call(
    _head_body,
    out_shape=(
        jax.ShapeDtypeStruct((B, R), jnp.float32),
        jax.ShapeDtypeStruct((B, 1), jnp.float32),
        jax.ShapeDtypeStruct((B, 1), jnp.float32),
    ),
)


def kernel(x, Wr, W_self, b_enc, W_fc, b_fc, W_dir, b_dir, W_cont, b_cont,
           edge_index, edge_type, nest_tensor, food_tensor):
  src = edge_index[0].astype(jnp.int32)
  dst = edge_index[1].astype(jnp.int32)
  typ = edge_type.astype(jnp.int32)
  sb, dg, xu = _sc_gather_scatter(src, dst, typ,
                                  nest_tensor.astype(jnp.int32),
                                  food_tensor.astype(jnp.int32), x)
  logits, mu, logvar = _head(
      sb, dg, xu, Wr, W_self, b_enc.reshape(1, D), W_fc, b_fc.reshape(1, -1),
      W_dir, b_dir.reshape(1, -1), W_cont, b_cont.reshape(1, -1))
  return logits, mu, logvar


# speedup vs baseline: 32.1849x; 1.0041x over previous
import jax
import jax.numpy as jnp
from jax import lax
from jax.experimental import pallas as pl
from jax.experimental.pallas import tpu as pltpu
from jax.experimental.pallas import tpu_sc as plsc

NC = 2
NS = 16
NW = NC * NS
L = 16

N = 10000
E = 320000
D = 128
R = 4
B = 1024
U = 2 * B
QROWS = U * R
QROWS_C = U * 2
TRASH = QROWS_C
SROWS_C = QROWS_C + 128
NPAD = 10240
EPS = E // NS
NHALF = 5
NSTRIP = 2
STRIP = EPS // (NHALF * NSTRIP)
CAP = 4096
KB = 64


def _sc_body(src_hbm, dst_hbm, typ_hbm, nest_hbm, food_hbm, x_hbm,
             sb_hbm, dg_hbm, xu_hbm,
             es_v, ed_v, et_v, slotmap, u_v, srcc, combc, comb2d, rows2,
             ones_v, slb_v, s_sh, deg_sh, sem_g, sem_s, sem_d):
  rows_v = rows2.at[0]
  cid = lax.axis_index("c")
  sid = lax.axis_index("s")
  wid = sid * NC + cid
  i16 = lax.broadcasted_iota(jnp.int32, (L,), 0)

  pltpu.sync_copy(nest_hbm, u_v.at[pl.ds(0, B)])
  pltpu.sync_copy(food_hbm, u_v.at[pl.ds(B, B)])

  def zrows(i, c):
    for cc in range(D // L):
      rows2[0, i, pl.ds(cc * L, L)] = jnp.zeros((L,), jnp.float32)
      ones_v[i, pl.ds(cc * L, L)] = jnp.full((L,), 1.0, jnp.float32)
    return c
  lax.fori_loop(0, KB, zrows, 0)

  s0 = sid * (SROWS_C // NS)
  zcopies = []
  for off, nn in ((0, KB), (KB, KB), (2 * KB, KB), (3 * KB, KB),
                  (4 * KB, 8)):
    zcopies.append(pltpu.async_copy(rows_v.at[pl.ds(0, nn)],
                                    s_sh.at[pl.ds(s0 + off, nn)], sem_s))
    zcopies.append(pltpu.async_copy(rows_v.at[pl.ds(0, nn)],
                                    deg_sh.at[pl.ds(s0 + off, nn)], sem_d))

  def sm_init(g, c):
    slotmap[pl.ds(g * L, L)] = jnp.full((L,), -1, jnp.int32)
    return c
  lax.fori_loop(0, NPAD // L, sm_init, 0)

  def sm_scat(g, c):
    idx = u_v[pl.ds(g * L, L)]
    plsc.store_scatter(slotmap, [idx], g * L + i16)
    return c
  lax.fori_loop(0, U // L, sm_scat, 0)

  for zc in zcopies:
    zc.wait()
  plsc.subcore_barrier()

  for half in range(NHALF):
    def strip_body(s, cnt):
      base = sid * EPS + (half * NSTRIP + s) * STRIP
      e1 = pltpu.async_copy(src_hbm.at[pl.ds(base, STRIP)], es_v, sem_g)
      e2 = pltpu.async_copy(dst_hbm.at[pl.ds(base, STRIP)], ed_v, sem_g)
      e3 = pltpu.async_copy(typ_hbm.at[pl.ds(base, STRIP)], et_v, sem_g)
      e1.wait()
      e2.wait()
      e3.wait()

      def grp(g, cnt):
        d16 = ed_v[pl.ds(g * L, L)]
        t16 = et_v[pl.ds(g * L, L)]
        s16 = es_v[pl.ds(g * L, L)]
        sl = plsc.load_gather(slotmap, [d16])
        m = jnp.logical_and(sl >= 0,
                            lax.shift_right_logical(t16, 1) == cid)
        comb = sl * 2 + lax.bitwise_and(t16, 1)
        plsc.store_compressed(srcc.at[pl.ds(cnt, L)], s16, mask=m)
        plsc.store_compressed(combc.at[pl.ds(cnt, L)], comb, mask=m)
        pc = plsc.all_reduce_population_count(m)
        return cnt + pc.max().astype(jnp.int32)
      return lax.fori_loop(0, STRIP // L, grp, cnt)
    cnt = lax.fori_loop(0, NSTRIP, strip_body, jnp.int32(0))

    def padt(t, c):
      srcc[pl.ds(cnt + t * L, L)] = jnp.zeros((L,), jnp.int32)
      combc[pl.ds(cnt + t * L, L)] = jnp.full((L,), TRASH, jnp.int32)
      return c
    lax.fori_loop(0, KB // L, padt, 0)
    nbat = lax.shift_right_logical(cnt + KB - 1, 6)

    def repack(g, c):
      comb2d[lax.shift_right_logical(g, 2),
             pl.ds(lax.bitwise_and(g, 3) * L, L)] = combc[pl.ds(g * L, L)]
      return c
    lax.fori_loop(0, nbat * (KB // L), repack, 0)

    @pl.when(nbat > 0)
    def _prime():
      pltpu.async_copy(x_hbm.at[srcc.at[pl.ds(0, KB)]], rows2.at[0], sem_g)

    def bat(bi, c):
      b = lax.bitwise_and(bi, 1)
      cur = rows2.at[b]
      pltpu.make_async_copy(x_hbm.at[srcc.at[pl.ds(bi * KB, KB)]], cur,
                            sem_g).wait()

      @pl.when(bi >= 1)
      def _drain_prev():
        prev = rows2.at[1 - b]
        pltpu.make_async_copy(prev, s_sh.at[comb2d.at[bi - 1]], sem_s).wait()
        pltpu.make_async_copy(ones_v, deg_sh.at[comb2d.at[bi - 1]],
                              sem_d).wait()

      @pl.when(bi + 1 < nbat)
      def _next_gather():
        pltpu.async_copy(x_hbm.at[srcc.at[pl.ds((bi + 1) * KB, KB)]],
                         rows2.at[1 - b], sem_g)

      pltpu.async_copy(cur, s_sh.at[comb2d.at[bi]], sem_s, add=True)
      pltpu.async_copy(ones_v, deg_sh.at[comb2d.at[bi]], sem_d, add=True)
      return c
    lax.fori_loop(0, nbat, bat, 0)

    @pl.when(nbat > 0)
    def _drain_last():
      bl = nbat - 1
      pltpu.make_async_copy(rows2.at[lax.bitwise_and(bl, 1)],
                            s_sh.at[comb2d.at[bl]], sem_s).wait()
      pltpu.make_async_copy(ones_v, deg_sh.at[comb2d.at[bl]], sem_d).wait()

  plsc.subcore_barrier()

  qbase = sid * (QROWS_C // NS)
  def slb_b(g, c):
    q = qbase + g * L + i16
    rp = lax.shift_right_logical(q, 11)
    j = lax.bitwise_and(q, 2047)
    uj = plsc.load_gather(u_v, [j])
    sl = plsc.load_gather(slotmap, [uj])
    slb_v[pl.ds(g * L, L)] = sl * 2 + rp
    return c
  lax.fori_loop(0, (QROWS_C // NS) // L, slb_b, 0)

  for k in range(4):
    idxs = slb_v.at[pl.ds(k * KB, KB)]
    gs = pltpu.async_copy(s_sh.at[idxs], rows2.at[0], sem_g)
    gd = pltpu.async_copy(deg_sh.at[idxs], rows2.at[1], sem_d)
    gs.wait()
    pltpu.sync_copy(rows2.at[0],
                    sb_hbm.at[pl.ds(cid * QROWS_C + qbase + k * KB, KB)])
    gd.wait()
    pltpu.sync_copy(rows2.at[1],
                    dg_hbm.at[pl.ds(cid * QROWS_C + qbase + k * KB, KB)])

  xo = wid * (U // NW)
  g1 = pltpu.async_copy(x_hbm.at[u_v.at[pl.ds(xo, 32)]],
                        rows2.at[0, pl.ds(0, 32)], sem_g)
  g2 = pltpu.async_copy(x_hbm.at[u_v.at[pl.ds(xo + 32, 32)]],
                        rows2.at[1, pl.ds(0, 32)], sem_d)
  g1.wait()
  pltpu.sync_copy(rows2.at[0, pl.ds(0, 32)], xu_hbm.at[pl.ds(xo, 32)])
  g2.wait()
  pltpu.sync_copy(rows2.at[1, pl.ds(0, 32)], xu_hbm.at[pl.ds(xo + 32, 32)])


_sc_gather_scatter = pl.kernel(
    _sc_body,
    out_type=(
        jax.ShapeDtypeStruct((QROWS, D), jnp.float32),
        jax.ShapeDtypeStruct((QROWS, D), jnp.float32),
        jax.ShapeDtypeStruct((U, D), jnp.float32),
    ),
    mesh=plsc.VectorSubcoreMesh(core_axis_name="c", subcore_axis_name="s"),
    compiler_params=pltpu.CompilerParams(needs_layout_passes=False),
    scratch_types=[
        pltpu.VMEM((STRIP,), jnp.int32),
        pltpu.VMEM((STRIP,), jnp.int32),
        pltpu.VMEM((STRIP,), jnp.int32),
        pltpu.VMEM((NPAD,), jnp.int32),
        pltpu.VMEM((U,), jnp.int32),
        pltpu.VMEM((CAP,), jnp.int32),
        pltpu.VMEM((CAP,), jnp.int32),
        pltpu.VMEM((CAP // KB, KB), jnp.int32),
        pltpu.VMEM((2, KB, D), jnp.float32),
        pltpu.VMEM((KB, D), jnp.float32),
        pltpu.VMEM((QROWS_C // NS,), jnp.int32),
        pltpu.VMEM_SHARED((SROWS_C, D), jnp.float32),
        pltpu.VMEM_SHARED((SROWS_C, D), jnp.float32),
        pltpu.SemaphoreType.DMA,
        pltpu.SemaphoreType.DMA,
        pltpu.SemaphoreType.DMA,
    ],
)


def _head_body(sb_ref, dg_ref, xu_ref, wr_ref, wself_ref, benc_ref, wfc_ref,
               bfc_ref, wdir_ref, bdir_ref, wcont_ref, bcont_ref,
               logits_ref, mu_ref, logvar_ref):
  f32 = jnp.float32
  hi = jax.lax.Precision.HIGHEST
  scale = 1.0 / jnp.maximum(dg_ref[:, 0:1], 1.0)
  sn = sb_ref[...] * scale
  contrib = jnp.zeros((U, D), f32)
  for r in range(R):
    contrib = contrib + jnp.dot(sn[r * U:(r + 1) * U], wr_ref[r],
                                preferred_element_type=f32, precision=hi)
  node = contrib + jnp.dot(xu_ref[...], wself_ref[...],
                           preferred_element_type=f32, precision=hi)
  node = jnp.maximum(node + benc_ref[...], 0.0)
  hin = jnp.concatenate([node[:B], node[B:]], axis=1)
  h = jnp.tanh(jnp.dot(hin, wfc_ref[...], preferred_element_type=f32,
                       precision=hi) + bfc_ref[...])
  logits_ref[...] = jnp.dot(h, wdir_ref[...], preferred_element_type=f32,
                            precision=hi) + bdir_ref[...]
  cont = jnp.dot(h, wcont_ref[...], preferred_element_type=f32,
                 precision=hi) + bcont_ref[...]
  mu_ref[...] = cont[:, 0:1]
  logvar_ref[...] = cont[:, 1:2]


_head = pl.pallas_call(
    _head_body,
    out_shape=(
        jax.ShapeDtypeStruct((B, R), jnp.float32),
        jax.ShapeDtypeStruct((B, 1), jnp.float32),
        jax.ShapeDtypeStruct((B, 1), jnp.float32),
    ),
)


def kernel(x, Wr, W_self, b_enc, W_fc, b_fc, W_dir, b_dir, W_cont, b_cont,
           edge_index, edge_type, nest_tensor, food_tensor):
  src = edge_index[0].astype(jnp.int32)
  dst = edge_index[1].astype(jnp.int32)
  typ = edge_type.astype(jnp.int32)
  sb, dg, xu = _sc_gather_scatter(src, dst, typ,
                                  nest_tensor.astype(jnp.int32),
                                  food_tensor.astype(jnp.int32), x)
  logits, mu, logvar = _head(
      sb, dg, xu, Wr, W_self, b_enc.reshape(1, D), W_fc, b_fc.reshape(1, -1),
      W_dir, b_dir.reshape(1, -1), W_cont, b_cont.reshape(1, -1))
  return logits, mu, logvar
